# speedup vs baseline: 4.6124x; 4.6124x over previous
import jax
import jax.numpy as jnp
from jax.experimental import pallas as pl

_VOCAB = 16
_HIDDEN = 128
_BLK = 8192


def _emb_kernel(ids_ref, emb_ref, out_ref):
    ids = ids_ref[...]
    iota = jax.lax.broadcasted_iota(jnp.int32, (1, _VOCAB), 1)
    one_hot = (ids == iota).astype(jnp.float32)
    out_ref[...] = jax.lax.dot_general(
        one_hot, emb_ref[...],
        (((1,), (0,)), ((), ())),
        preferred_element_type=jnp.float32,
    )


def kernel(input_ids, embedding):
    batch, seq = input_ids.shape
    n = batch * seq
    ids_col = input_ids.reshape(n, 1).astype(jnp.int32)
    grid = (n // _BLK,)
    out = pl.pallas_call(
        _emb_kernel,
        grid=grid,
        in_specs=[
            pl.BlockSpec((_BLK, 1), lambda i: (i, 0)),
            pl.BlockSpec((_VOCAB, _HIDDEN), lambda i: (0, 0)),
        ],
        out_specs=pl.BlockSpec((_BLK, _HIDDEN), lambda i: (i, 0)),
        out_shape=jax.ShapeDtypeStruct((n, _HIDDEN), jnp.float32),
    )(ids_col, embedding)
    hidden = out.reshape(batch, seq, _HIDDEN)
    return (hidden, hidden)


# speedup vs baseline: 5.6845x; 1.2324x over previous
import jax
import jax.numpy as jnp
from jax.experimental import pallas as pl

_VOCAB = 16
_HIDDEN = 128
_BLK = 8192


def _emb_kernel(ids_ref, emb_ref, out0_ref, out1_ref):
    ids = ids_ref[...]
    iota = jax.lax.broadcasted_iota(jnp.int32, (1, _VOCAB), 1)
    one_hot = (ids == iota).astype(jnp.float32)
    res = jax.lax.dot_general(
        one_hot, emb_ref[...],
        (((1,), (0,)), ((), ())),
        preferred_element_type=jnp.float32,
    )
    out0_ref[...] = res
    out1_ref[...] = res


def kernel(input_ids, embedding):
    batch, seq = input_ids.shape
    n = batch * seq
    ids_col = input_ids.reshape(n, 1).astype(jnp.int32)
    grid = (n // _BLK,)
    out_spec = pl.BlockSpec((_BLK, _HIDDEN), lambda i: (i, 0))
    out_shape = jax.ShapeDtypeStruct((n, _HIDDEN), jnp.float32)
    out0, out1 = pl.pallas_call(
        _emb_kernel,
        grid=grid,
        in_specs=[
            pl.BlockSpec((_BLK, 1), lambda i: (i, 0)),
            pl.BlockSpec((_VOCAB, _HIDDEN), lambda i: (0, 0)),
        ],
        out_specs=(out_spec, out_spec),
        out_shape=(out_shape, out_shape),
    )(ids_col, embedding)
    return (out0.reshape(batch, seq, _HIDDEN), out1.reshape(batch, seq, _HIDDEN))


# speedup vs baseline: 5.7736x; 1.0157x over previous
import jax
import jax.numpy as jnp
from jax.experimental import pallas as pl

_VOCAB = 16
_HIDDEN = 128
_BLK = 16384


def _emb_kernel(ids_ref, emb_ref, out0_ref, out1_ref):
    ids = ids_ref[...]
    iota = jax.lax.broadcasted_iota(jnp.int32, (1, _VOCAB), 1)
    one_hot = (ids == iota).astype(jnp.float32)
    res = jax.lax.dot_general(
        one_hot, emb_ref[...],
        (((1,), (0,)), ((), ())),
        preferred_element_type=jnp.float32,
    )
    out0_ref[...] = res
    out1_ref[...] = res


def kernel(input_ids, embedding):
    batch, seq = input_ids.shape
    n = batch * seq
    ids_col = input_ids.reshape(n, 1).astype(jnp.int32)
    grid = (n // _BLK,)
    out_spec = pl.BlockSpec((_BLK, _HIDDEN), lambda i: (i, 0))
    out_shape = jax.ShapeDtypeStruct((n, _HIDDEN), jnp.float32)
    out0, out1 = pl.pallas_call(
        _emb_kernel,
        grid=grid,
        in_specs=[
            pl.BlockSpec((_BLK, 1), lambda i: (i, 0)),
            pl.BlockSpec((_VOCAB, _HIDDEN), lambda i: (0, 0)),
        ],
        out_specs=(out_spec, out_spec),
        out_shape=(out_shape, out_shape),
    )(ids_col, embedding)
    return (out0.reshape(batch, seq, _HIDDEN), out1.reshape(batch, seq, _HIDDEN))
